# baseline (speedup 1.0000x reference)
import jax
import jax.numpy as jnp
from jax import lax
from jax.experimental import pallas as pl
from jax.experimental.pallas import tpu as pltpu

_N = 8192
_FEAT = 128
_HID = 64
_EMB = 32

_BM = 256
_DM = 512
_DN = 2048


def _proj1_body(h_ref, w0_ref, w1_ref, hw0_ref, hw1_ref):
    h = h_ref[...]
    dims = (((1,), (1,)), ((), ()))
    hw0_ref[...] = lax.dot_general(h, w0_ref[...], dims,
                                   preferred_element_type=jnp.float32)
    hw1_ref[...] = lax.dot_general(h, w1_ref[...], dims,
                                   preferred_element_type=jnp.float32)


def _pass1_body(a0_ref, a1_ref, hw0_ref, hw1_ref, b1_ref, w20_ref, w21_ref,
                g0_ref, g1_ref):
    mm = (((1,), (0,)), ((), ()))
    acc = lax.dot_general(a0_ref[...], hw0_ref[...], mm,
                          preferred_element_type=jnp.float32)
    acc = acc + lax.dot_general(a1_ref[...], hw1_ref[...], mm,
                                preferred_element_type=jnp.float32)
    h1 = jnp.maximum(acc + b1_ref[...], 0.0)
    dims = (((1,), (1,)), ((), ()))
    g0_ref[...] = lax.dot_general(h1, w20_ref[...], dims,
                                  preferred_element_type=jnp.float32)
    g1_ref[...] = lax.dot_general(h1, w21_ref[...], dims,
                                  preferred_element_type=jnp.float32)


def _pass2_body(a0_ref, a1_ref, g0_ref, g1_ref, b2_ref, z_ref):
    mm = (((1,), (0,)), ((), ()))
    acc = lax.dot_general(a0_ref[...], g0_ref[...], mm,
                          preferred_element_type=jnp.float32)
    acc = acc + lax.dot_general(a1_ref[...], g1_ref[...], mm,
                                preferred_element_type=jnp.float32)
    z_ref[...] = acc + b2_ref[...]


def _decode_body(zi_ref, zj_ref, out_ref):
    dims = (((1,), (1,)), ((), ()))
    out_ref[...] = lax.dot_general(zi_ref[...], zj_ref[...], dims,
                                   preferred_element_type=jnp.float32)


def kernel(H, A_norm_r0, A_norm_r1, W1_r0, W1_r1, b1, W2_r0, W2_r1, b2):
    b1_2d = b1.reshape(1, _HID)
    b2_2d = b2.reshape(1, _EMB)

    full = lambda shape: pl.BlockSpec(shape, lambda i: (0, 0))

    hw0, hw1 = pl.pallas_call(
        _proj1_body,
        grid=(1,),
        in_specs=[full((_N, _FEAT)), full((_HID, _FEAT)), full((_HID, _FEAT))],
        out_specs=[full((_N, _HID)), full((_N, _HID))],
        out_shape=[jax.ShapeDtypeStruct((_N, _HID), jnp.float32)] * 2,
    )(H, W1_r0, W1_r1)

    g0, g1 = pl.pallas_call(
        _pass1_body,
        grid=(_N // _BM,),
        in_specs=[
            pl.BlockSpec((_BM, _N), lambda i: (i, 0)),
            pl.BlockSpec((_BM, _N), lambda i: (i, 0)),
            full((_N, _HID)),
            full((_N, _HID)),
            full((1, _HID)),
            full((_EMB, _HID)),
            full((_EMB, _HID)),
        ],
        out_specs=[
            pl.BlockSpec((_BM, _EMB), lambda i: (i, 0)),
            pl.BlockSpec((_BM, _EMB), lambda i: (i, 0)),
        ],
        out_shape=[jax.ShapeDtypeStruct((_N, _EMB), jnp.float32)] * 2,
        compiler_params=pltpu.CompilerParams(
            dimension_semantics=("arbitrary",)),
    )(A_norm_r0, A_norm_r1, hw0, hw1, b1_2d, W2_r0, W2_r1)

    z = pl.pallas_call(
        _pass2_body,
        grid=(_N // _BM,),
        in_specs=[
            pl.BlockSpec((_BM, _N), lambda i: (i, 0)),
            pl.BlockSpec((_BM, _N), lambda i: (i, 0)),
            full((_N, _EMB)),
            full((_N, _EMB)),
            full((1, _EMB)),
        ],
        out_specs=pl.BlockSpec((_BM, _EMB), lambda i: (i, 0)),
        out_shape=jax.ShapeDtypeStruct((_N, _EMB), jnp.float32),
        compiler_params=pltpu.CompilerParams(
            dimension_semantics=("arbitrary",)),
    )(A_norm_r0, A_norm_r1, g0, g1, b2_2d)

    a_hat = pl.pallas_call(
        _decode_body,
        grid=(_N // _DM, _N // _DN),
        in_specs=[
            pl.BlockSpec((_DM, _EMB), lambda i, j: (i, 0)),
            pl.BlockSpec((_DN, _EMB), lambda i, j: (j, 0)),
        ],
        out_specs=pl.BlockSpec((_DM, _DN), lambda i, j: (i, j)),
        out_shape=jax.ShapeDtypeStruct((_N, _N), jnp.float32),
        compiler_params=pltpu.CompilerParams(
            dimension_semantics=("parallel", "parallel")),
    )(z, z)

    return (z, a_hat)


# speedup vs baseline: 1.0604x; 1.0604x over previous
import jax
import jax.numpy as jnp
from jax import lax
from jax.experimental import pallas as pl
from jax.experimental.pallas import tpu as pltpu

_N = 8192
_FEAT = 128
_HID = 64
_EMB = 32

_BM = 256
_DM = 256


def _proj1_body(h_ref, w0_ref, w1_ref, hw0_ref, hw1_ref):
    h = h_ref[...]
    dims = (((1,), (1,)), ((), ()))
    hw0_ref[...] = lax.dot_general(h, w0_ref[...], dims,
                                   preferred_element_type=jnp.float32)
    hw1_ref[...] = lax.dot_general(h, w1_ref[...], dims,
                                   preferred_element_type=jnp.float32)


def _pass1_body(a0_ref, a1_ref, hw0_ref, hw1_ref, b1_ref, w20_ref, w21_ref,
                g0_ref, g1_ref):
    mm = (((1,), (0,)), ((), ()))
    acc = lax.dot_general(a0_ref[...], hw0_ref[...], mm,
                          preferred_element_type=jnp.float32)
    acc = acc + lax.dot_general(a1_ref[...], hw1_ref[...], mm,
                                preferred_element_type=jnp.float32)
    h1 = jnp.maximum(acc + b1_ref[...], 0.0)
    dims = (((1,), (1,)), ((), ()))
    g0_ref[...] = lax.dot_general(h1, w20_ref[...], dims,
                                  preferred_element_type=jnp.float32)
    g1_ref[...] = lax.dot_general(h1, w21_ref[...], dims,
                                  preferred_element_type=jnp.float32)


def _pass2_body(a0_ref, a1_ref, g0_ref, g1_ref, b2_ref, z_ref):
    mm = (((1,), (0,)), ((), ()))
    acc = lax.dot_general(a0_ref[...], g0_ref[...], mm,
                          preferred_element_type=jnp.float32)
    acc = acc + lax.dot_general(a1_ref[...], g1_ref[...], mm,
                                preferred_element_type=jnp.float32)
    z_ref[...] = acc + b2_ref[...]


def _decode_body(zi_ref, zj_ref, out_ref):
    dims = (((1,), (1,)), ((), ()))
    out_ref[...] = lax.dot_general(zi_ref[...], zj_ref[...], dims,
                                   preferred_element_type=jnp.float32)


def kernel(H, A_norm_r0, A_norm_r1, W1_r0, W1_r1, b1, W2_r0, W2_r1, b2):
    b1_2d = b1.reshape(1, _HID)
    b2_2d = b2.reshape(1, _EMB)

    full = lambda shape: pl.BlockSpec(shape, lambda i: (0, 0))

    hw0, hw1 = pl.pallas_call(
        _proj1_body,
        grid=(8,),
        in_specs=[
            pl.BlockSpec((_N // 8, _FEAT), lambda i: (i, 0)),
            full((_HID, _FEAT)),
            full((_HID, _FEAT)),
        ],
        out_specs=[
            pl.BlockSpec((_N // 8, _HID), lambda i: (i, 0)),
            pl.BlockSpec((_N // 8, _HID), lambda i: (i, 0)),
        ],
        out_shape=[jax.ShapeDtypeStruct((_N, _HID), jnp.float32)] * 2,
        compiler_params=pltpu.CompilerParams(
            dimension_semantics=("parallel",)),
    )(H, W1_r0, W1_r1)

    g0, g1 = pl.pallas_call(
        _pass1_body,
        grid=(_N // _BM,),
        in_specs=[
            pl.BlockSpec((_BM, _N), lambda i: (i, 0)),
            pl.BlockSpec((_BM, _N), lambda i: (i, 0)),
            full((_N, _HID)),
            full((_N, _HID)),
            full((1, _HID)),
            full((_EMB, _HID)),
            full((_EMB, _HID)),
        ],
        out_specs=[
            pl.BlockSpec((_BM, _EMB), lambda i: (i, 0)),
            pl.BlockSpec((_BM, _EMB), lambda i: (i, 0)),
        ],
        out_shape=[jax.ShapeDtypeStruct((_N, _EMB), jnp.float32)] * 2,
        compiler_params=pltpu.CompilerParams(
            dimension_semantics=("parallel",)),
    )(A_norm_r0, A_norm_r1, hw0, hw1, b1_2d, W2_r0, W2_r1)

    z = pl.pallas_call(
        _pass2_body,
        grid=(_N // _BM,),
        in_specs=[
            pl.BlockSpec((_BM, _N), lambda i: (i, 0)),
            pl.BlockSpec((_BM, _N), lambda i: (i, 0)),
            full((_N, _EMB)),
            full((_N, _EMB)),
            full((1, _EMB)),
        ],
        out_specs=pl.BlockSpec((_BM, _EMB), lambda i: (i, 0)),
        out_shape=jax.ShapeDtypeStruct((_N, _EMB), jnp.float32),
        compiler_params=pltpu.CompilerParams(
            dimension_semantics=("parallel",)),
    )(A_norm_r0, A_norm_r1, g0, g1, b2_2d)

    a_hat = pl.pallas_call(
        _decode_body,
        grid=(_N // _DM,),
        in_specs=[
            pl.BlockSpec((_DM, _EMB), lambda i: (i, 0)),
            full((_N, _EMB)),
        ],
        out_specs=pl.BlockSpec((_DM, _N), lambda i: (i, 0)),
        out_shape=jax.ShapeDtypeStruct((_N, _N), jnp.float32),
        compiler_params=pltpu.CompilerParams(
            dimension_semantics=("parallel",)),
    )(z, z)

    return (z, a_hat)
